# Initial kernel scaffold; baseline (speedup 1.0000x reference)
import jax
import jax.numpy as jnp
from jax.experimental import pallas as pl


def kernel(x, pp_W1, pp_b1, fc_W1, mu1, inv_sigma1, gbias1, pp_W2, pp_b2, fc_W2, mu2, inv_sigma2, gbias2, edge_index):
    raise NotImplementedError("write your pallas kernel here")


# speedup vs baseline: 7.4621x; 7.4621x over previous
import dataclasses
import functools

import jax
import jax.numpy as jnp
from jax import lax
from jax.experimental import pallas as pl
from jax.experimental.pallas import tpu as pltpu
from jax.experimental.pallas import tpu_sc as plsc

N = 10000
NP = 10240
E = 320000
F = 128
HF = F // 2
K = 3
D = 2
L = 16
RW = K * HF
CH = 16
EPT = E // 16
NCH = EPT // CH
STRIPE = NP // 16


def _cparams():
    cp = pltpu.CompilerParams()
    fields = pltpu.CompilerParams.__dataclass_fields__
    if "needs_layout_passes" in fields:
        cp = dataclasses.replace(cp, needs_layout_passes=False)
    if "use_tc_tiling_on_sc" in fields:
        cp = dataclasses.replace(cp, use_tc_tiling_on_sc=False)
    return cp


def _vc(v, dtype=jnp.float32):
    return jnp.full((L,), v, dtype)


def _sc_degree(dst3):
    nch = dst3.shape[1]
    mesh = plsc.VectorSubcoreMesh(core_axis_name="c", subcore_axis_name="s")

    @functools.partial(
        pl.kernel,
        out_type=jax.ShapeDtypeStruct((2, NP), jnp.float32),
        mesh=mesh,
        compiler_params=_cparams(),
        scratch_types=[
            pltpu.VMEM((nch, CH), jnp.int32),
            pltpu.VMEM((L,), jnp.float32),
            pltpu.VMEM((L,), jnp.float32),
            pltpu.VMEM_SHARED((NP,), jnp.float32),
        ],
    )
    def kfn(dst_hbm, deg_hbm, dst_v, one_v, zer_v, deg_sh):
        cid = lax.axis_index("c")
        sid = lax.axis_index("s")
        wid = cid * 16 + sid
        pltpu.sync_copy(dst_hbm.at[wid], dst_v)
        one_v[...] = _vc(1.0)
        zer_v[...] = _vc(0.0)

        @pl.loop(0, STRIPE // L)
        def _(t):
            pltpu.sync_copy(zer_v, deg_sh.at[pl.ds(sid * STRIPE + t * L, L)])

        plsc.subcore_barrier()

        @pl.loop(0, nch)
        def _(c):
            pltpu.sync_copy(one_v, deg_sh.at[dst_v.at[c]], add=True)

        plsc.subcore_barrier()
        pltpu.sync_copy(deg_sh.at[pl.ds(sid * STRIPE, STRIPE)],
                        deg_hbm.at[cid, pl.ds(sid * STRIPE, STRIPE)])

    return kfn(dst3)


def _sc_msg(proj2h, rsq, src3, dst3, par):
    mesh = plsc.VectorSubcoreMesh(core_axis_name="c", subcore_axis_name="s")

    @functools.partial(
        pl.kernel,
        out_type=jax.ShapeDtypeStruct((2, NP, HF), jnp.float32),
        mesh=mesh,
        compiler_params=_cparams(),
        scratch_types=[
            pltpu.VMEM((NCH, CH), jnp.int32),
            pltpu.VMEM((NCH, CH), jnp.int32),
            pltpu.VMEM((NP,), jnp.float32),
            pltpu.VMEM((18, L), jnp.float32),
            pltpu.VMEM((2, CH, RW), jnp.float32),
            pltpu.VMEM((CH, HF), jnp.float32),
            pltpu.VMEM((K, L), jnp.float32),
            pltpu.VMEM_SHARED((NP, HF), jnp.float32),
            pltpu.SemaphoreType.DMA,
            pltpu.SemaphoreType.DMA,
        ],
    )
    def kfn(proj_hbm, rsq_hbm, src_hbm, dst_hbm, par_hbm, out_hbm,
            src_v, dst_v, rsq_v, par_v, row_v, msg_v, gw_v, agg_sh,
            sem0, sem1):
        cid = lax.axis_index("c")
        sid = lax.axis_index("s")
        pltpu.sync_copy(src_hbm.at[sid], src_v)
        pltpu.sync_copy(dst_hbm.at[sid], dst_v)
        pltpu.sync_copy(rsq_hbm, rsq_v)
        pltpu.sync_copy(par_hbm, par_v)

        for i in range(CH):
            for j in range(HF // L):
                msg_v[i, pl.ds(j * L, L)] = _vc(0.0)

        @pl.loop(0, STRIPE // CH)
        def _(t):
            pltpu.sync_copy(msg_v, agg_sh.at[pl.ds(sid * STRIPE + t * CH, CH)])

        plsc.subcore_barrier()

        pA = [par_v[0, :], par_v[1, :]]
        pB = [par_v[2, :], par_v[3, :]]
        pC = [par_v[4, :], par_v[5, :]]
        pMU = [[par_v[6 + 2 * k + d_, :] for d_ in range(D)] for k in range(K)]
        pIS = [[par_v[12 + 2 * k + d_, :] for d_ in range(D)] for k in range(K)]

        def compute_chunk(c, buf):
            s16 = src_v[c, :]
            d16 = dst_v[c, :]
            ru = plsc.load_gather(rsq_v, [s16])
            rv = plsc.load_gather(rsq_v, [d16])
            ps = []
            for d_ in range(D):
                z = pA[d_] * ru + pB[d_] * rv + pC[d_]
                az = jnp.abs(z)
                e2 = jnp.exp(az + az)
                t = _vc(1.0) - _vc(2.0) / (e2 + _vc(1.0))
                ps.append(jnp.sign(z) * t)
            for k in range(K):
                u0 = (ps[0] - pMU[k][0]) * pIS[k][0]
                u1 = (ps[1] - pMU[k][1]) * pIS[k][1]
                gw_v[k, :] = jnp.exp(_vc(-0.5) * (u0 * u0 + u1 * u1))
            for i in range(CH):
                ii = jnp.full((L,), i, jnp.int32)
                g0 = plsc.load_gather(gw_v.at[0], [ii])
                g1 = plsc.load_gather(gw_v.at[1], [ii])
                g2 = plsc.load_gather(gw_v.at[2], [ii])
                for j in range(HF // L):
                    v0 = row_v[buf, i, pl.ds(j * L, L)]
                    v1 = row_v[buf, i, pl.ds(HF + j * L, L)]
                    v2 = row_v[buf, i, pl.ds(2 * HF + j * L, L)]
                    msg_v[i, pl.ds(j * L, L)] = v0 * g0 + v1 * g1 + v2 * g2
            pltpu.sync_copy(msg_v, agg_sh.at[dst_v.at[c]], add=True)

        def gather(c, buf, sem):
            return pltpu.async_copy(proj_hbm.at[cid].at[src_v.at[c]],
                                    row_v.at[buf], sem)

        def wait_gather(c, buf, sem):
            pltpu.make_async_copy(proj_hbm.at[cid].at[src_v.at[c]],
                                  row_v.at[buf], sem).wait()

        gather(0, 0, sem0)

        @pl.loop(0, NCH // 2 - 1)
        def _(p):
            c0 = p * 2
            gather(c0 + 1, 1, sem1)
            wait_gather(c0, 0, sem0)
            compute_chunk(c0, 0)
            gather(c0 + 2, 0, sem0)
            wait_gather(c0 + 1, 1, sem1)
            compute_chunk(c0 + 1, 1)

        gather(NCH - 1, 1, sem1)
        wait_gather(NCH - 2, 0, sem0)
        compute_chunk(NCH - 2, 0)
        wait_gather(NCH - 1, 1, sem1)
        compute_chunk(NCH - 1, 1)

        plsc.subcore_barrier()

        @pl.loop(0, STRIPE // 128)
        def _(t):
            pltpu.sync_copy(agg_sh.at[pl.ds(sid * STRIPE + t * 128, 128)],
                            out_hbm.at[cid, pl.ds(sid * STRIPE + t * 128, 128)])

    return kfn(proj2h, rsq, src3, dst3, par)


def _tc_rsq(degp):
    def body(dp_ref, o_ref):
        o_ref[...] = lax.rsqrt(dp_ref[0] + dp_ref[1])

    out = pl.pallas_call(
        body,
        out_shape=jax.ShapeDtypeStruct((NP // 128, 128), jnp.float32),
    )(degp.reshape(2, NP // 128, 128))
    return out.reshape(NP)


def _tc_proj(h, wre):
    def body(h_ref, w_ref, o_ref):
        hv = h_ref[...]
        o_ref[0] = lax.dot_general(hv, w_ref[0], (((1,), (1,)), ((), ())),
                                   preferred_element_type=jnp.float32)
        o_ref[1] = lax.dot_general(hv, w_ref[1], (((1,), (1,)), ((), ())),
                                   preferred_element_type=jnp.float32)

    return pl.pallas_call(
        body,
        out_shape=jax.ShapeDtypeStruct((2, h.shape[0], RW), jnp.float32),
    )(h, wre)


def _tc_sum_bias_proj(msgp, gb, wre):
    def body(mp_ref, b_ref, w_ref, o_ref):
        h = lax.concatenate([mp_ref[0], mp_ref[1]], 1) + b_ref[...]
        o_ref[0] = lax.dot_general(h, w_ref[0], (((1,), (1,)), ((), ())),
                                   preferred_element_type=jnp.float32)
        o_ref[1] = lax.dot_general(h, w_ref[1], (((1,), (1,)), ((), ())),
                                   preferred_element_type=jnp.float32)

    return pl.pallas_call(
        body,
        out_shape=jax.ShapeDtypeStruct((2, NP, RW), jnp.float32),
    )(msgp, gb.reshape(1, F), wre)


def _tc_final(msgp, gb):
    def body(mp_ref, b_ref, o_ref):
        o_ref[...] = lax.concatenate([mp_ref[0], mp_ref[1]], 1) + b_ref[...]

    return pl.pallas_call(
        body,
        out_shape=jax.ShapeDtypeStruct((N, F), jnp.float32),
    )(msgp[:, :N, :], gb.reshape(1, F))


def _pack_params(pp_W, pp_b, mu, inv_sigma):
    scal = jnp.concatenate([
        pp_W[:, 0], pp_W[:, 1], pp_b,
        mu.reshape(-1), inv_sigma.reshape(-1),
    ]).astype(jnp.float32)
    return jnp.broadcast_to(scal[:, None], (18, L))


def _reorder_w(fc_W):
    return fc_W.reshape(K, 2, HF, F).transpose(1, 0, 2, 3).reshape(2, RW, F)


def kernel(x, pp_W1, pp_b1, fc_W1, mu1, inv_sigma1, gbias1,
           pp_W2, pp_b2, fc_W2, mu2, inv_sigma2, gbias2, edge_index):
    src3 = edge_index[0].reshape(16, NCH, CH)
    dst3 = edge_index[1].reshape(16, NCH, CH)
    dst3_deg = edge_index[1].reshape(32, E // (32 * CH), CH)
    degp = _sc_degree(dst3_deg)
    rsq = _tc_rsq(degp)
    proj1 = _tc_proj(x, _reorder_w(fc_W1))
    msgp1 = _sc_msg(proj1, rsq, src3, dst3,
                    _pack_params(pp_W1, pp_b1, mu1, inv_sigma1))
    proj2 = _tc_sum_bias_proj(msgp1, gbias1, _reorder_w(fc_W2))
    msgp2 = _sc_msg(proj2, rsq, src3, dst3,
                    _pack_params(pp_W2, pp_b2, mu2, inv_sigma2))
    return _tc_final(msgp2, gbias2)


# speedup vs baseline: 10.9541x; 1.4680x over previous
import dataclasses
import functools

import jax
import jax.numpy as jnp
from jax import lax
from jax.experimental import pallas as pl
from jax.experimental.pallas import tpu as pltpu
from jax.experimental.pallas import tpu_sc as plsc

N = 10000
NP = 10240
E = 320000
F = 128
HF = F // 2
K = 3
D = 2
L = 16
RW = K * HF
CH = 32
CHD = 16
EPT = E // 16
NCH = EPT // CH
STRIPE = NP // 16


def _cparams():
    cp = pltpu.CompilerParams()
    fields = pltpu.CompilerParams.__dataclass_fields__
    if "needs_layout_passes" in fields:
        cp = dataclasses.replace(cp, needs_layout_passes=False)
    if "use_tc_tiling_on_sc" in fields:
        cp = dataclasses.replace(cp, use_tc_tiling_on_sc=False)
    return cp


def _vc(v, dtype=jnp.float32):
    return jnp.full((L,), v, dtype)


def _sc_degree(dst3):
    nch = dst3.shape[1]
    mesh = plsc.VectorSubcoreMesh(core_axis_name="c", subcore_axis_name="s")

    @functools.partial(
        pl.kernel,
        out_type=jax.ShapeDtypeStruct((2, NP), jnp.float32),
        mesh=mesh,
        compiler_params=_cparams(),
        scratch_types=[
            pltpu.VMEM((nch, CHD), jnp.int32),
            pltpu.VMEM((L,), jnp.float32),
            pltpu.VMEM((L,), jnp.float32),
            pltpu.VMEM_SHARED((NP,), jnp.float32),
        ],
    )
    def kfn(dst_hbm, deg_hbm, dst_v, one_v, zer_v, deg_sh):
        cid = lax.axis_index("c")
        sid = lax.axis_index("s")
        wid = cid * 16 + sid
        pltpu.sync_copy(dst_hbm.at[wid], dst_v)
        one_v[...] = _vc(1.0)
        zer_v[...] = _vc(0.0)

        @pl.loop(0, STRIPE // L)
        def _(t):
            pltpu.sync_copy(zer_v, deg_sh.at[pl.ds(sid * STRIPE + t * L, L)])

        plsc.subcore_barrier()

        @pl.loop(0, nch)
        def _(c):
            pltpu.sync_copy(one_v, deg_sh.at[dst_v.at[c]], add=True)

        plsc.subcore_barrier()
        pltpu.sync_copy(deg_sh.at[pl.ds(sid * STRIPE, STRIPE)],
                        deg_hbm.at[cid, pl.ds(sid * STRIPE, STRIPE)])

    return kfn(dst3)


def _sc_msg(proj2h, rsq, src3, dst3, par):
    mesh = plsc.VectorSubcoreMesh(core_axis_name="c", subcore_axis_name="s")

    @functools.partial(
        pl.kernel,
        out_type=jax.ShapeDtypeStruct((2, NP, HF), jnp.float32),
        mesh=mesh,
        compiler_params=_cparams(),
        scratch_types=[
            pltpu.VMEM((NCH, CH), jnp.int32),
            pltpu.VMEM((NCH, CH), jnp.int32),
            pltpu.VMEM((NP,), jnp.float32),
            pltpu.VMEM((18, L), jnp.float32),
            pltpu.VMEM((2, CH, RW), jnp.float32),
            pltpu.VMEM((2, CH, HF), jnp.float32),
            pltpu.VMEM((2, K, L), jnp.float32),
            pltpu.VMEM_SHARED((NP, HF), jnp.float32),
            pltpu.SemaphoreType.DMA,
            pltpu.SemaphoreType.DMA,
            pltpu.SemaphoreType.DMA,
            pltpu.SemaphoreType.DMA,
        ],
    )
    def kfn(proj_hbm, rsq_hbm, src_hbm, dst_hbm, par_hbm, out_hbm,
            src_v, dst_v, rsq_v, par_v, row_v, msg_v, gw_v, agg_sh,
            sem0, sem1, sct0, sct1):
        cid = lax.axis_index("c")
        sid = lax.axis_index("s")
        pltpu.sync_copy(src_hbm.at[sid], src_v)
        pltpu.sync_copy(dst_hbm.at[sid], dst_v)
        pltpu.sync_copy(rsq_hbm, rsq_v)
        pltpu.sync_copy(par_hbm, par_v)

        for i in range(CH):
            for j in range(HF // L):
                msg_v[0, i, pl.ds(j * L, L)] = _vc(0.0)

        @pl.loop(0, STRIPE // CH)
        def _(t):
            pltpu.sync_copy(msg_v.at[0],
                            agg_sh.at[pl.ds(sid * STRIPE + t * CH, CH)])

        plsc.subcore_barrier()

        pA = [par_v[0, :], par_v[1, :]]
        pB = [par_v[2, :], par_v[3, :]]
        pC = [par_v[4, :], par_v[5, :]]
        pMU = [[par_v[6 + 2 * k + d_, :] for d_ in range(D)] for k in range(K)]
        pIS = [[par_v[12 + 2 * k + d_, :] for d_ in range(D)] for k in range(K)]

        def compute_chunk(c, buf):
            for g in range(CH // L):
                s16 = src_v[c, pl.ds(g * L, L)]
                d16 = dst_v[c, pl.ds(g * L, L)]
                ru = plsc.load_gather(rsq_v, [s16])
                rv = plsc.load_gather(rsq_v, [d16])
                ps = []
                for d_ in range(D):
                    z = pA[d_] * ru + pB[d_] * rv + pC[d_]
                    az = jnp.abs(z)
                    e2 = jnp.exp(az + az)
                    t = _vc(1.0) - _vc(2.0) / (e2 + _vc(1.0))
                    ps.append(jnp.sign(z) * t)
                for k in range(K):
                    u0 = (ps[0] - pMU[k][0]) * pIS[k][0]
                    u1 = (ps[1] - pMU[k][1]) * pIS[k][1]
                    gw_v[g, k, :] = jnp.exp(_vc(-0.5) * (u0 * u0 + u1 * u1))
                for i in range(L):
                    ii = jnp.full((L,), i, jnp.int32)
                    g0 = plsc.load_gather(gw_v.at[g, 0], [ii])
                    g1 = plsc.load_gather(gw_v.at[g, 1], [ii])
                    g2 = plsc.load_gather(gw_v.at[g, 2], [ii])
                    e = g * L + i
                    for j in range(HF // L):
                        v0 = row_v[buf, e, pl.ds(j * L, L)]
                        v1 = row_v[buf, e, pl.ds(HF + j * L, L)]
                        v2 = row_v[buf, e, pl.ds(2 * HF + j * L, L)]
                        msg_v[buf, e, pl.ds(j * L, L)] = (
                            v0 * g0 + v1 * g1 + v2 * g2)

        def gather(c, buf, sem):
            pltpu.async_copy(proj_hbm.at[cid].at[src_v.at[c]],
                             row_v.at[buf], sem)

        def wait_gather(c, buf, sem):
            pltpu.make_async_copy(proj_hbm.at[cid].at[src_v.at[c]],
                                  row_v.at[buf], sem).wait()

        def scatter(c, buf, sem):
            pltpu.async_copy(msg_v.at[buf], agg_sh.at[dst_v.at[c]], sem,
                             add=True)

        def wait_scatter(c, buf, sem):
            pltpu.make_async_copy(msg_v.at[buf], agg_sh.at[dst_v.at[c]],
                                  sem).wait()

        gather(0, 0, sem0)
        gather(1, 1, sem1)

        @pl.loop(0, NCH // 2)
        def _(p):
            c0 = p * 2
            wait_gather(c0, 0, sem0)

            @pl.when(p >= 1)
            def _():
                wait_scatter(c0 - 2, 0, sct0)

            compute_chunk(c0, 0)
            scatter(c0, 0, sct0)
            gather(c0 + 2, 0, sem0)
            wait_gather(c0 + 1, 1, sem1)

            @pl.when(p >= 1)
            def _():
                wait_scatter(c0 - 1, 1, sct1)

            compute_chunk(c0 + 1, 1)
            scatter(c0 + 1, 1, sct1)

            @pl.when(p < NCH // 2 - 1)
            def _():
                gather(c0 + 3, 1, sem1)

        wait_gather(NCH - 1, 0, sem0)
        wait_scatter(NCH - 3, 0, sct0)
        compute_chunk(NCH - 1, 0)
        scatter(NCH - 1, 0, sct0)
        wait_scatter(NCH - 2, 1, sct1)
        wait_scatter(NCH - 1, 0, sct0)

        plsc.subcore_barrier()

        @pl.loop(0, STRIPE // 128)
        def _(t):
            pltpu.sync_copy(agg_sh.at[pl.ds(sid * STRIPE + t * 128, 128)],
                            out_hbm.at[cid, pl.ds(sid * STRIPE + t * 128, 128)])

    return kfn(proj2h, rsq, src3, dst3, par)


def _tc_rsq(degp):
    def body(dp_ref, o_ref):
        o_ref[...] = lax.rsqrt(dp_ref[0] + dp_ref[1])

    out = pl.pallas_call(
        body,
        out_shape=jax.ShapeDtypeStruct((NP // 128, 128), jnp.float32),
    )(degp.reshape(2, NP // 128, 128))
    return out.reshape(NP)


def _tc_proj(h, wre):
    def body(h_ref, w_ref, o_ref):
        hv = h_ref[...]
        o_ref[0] = lax.dot_general(hv, w_ref[0], (((1,), (1,)), ((), ())),
                                   preferred_element_type=jnp.float32)
        o_ref[1] = lax.dot_general(hv, w_ref[1], (((1,), (1,)), ((), ())),
                                   preferred_element_type=jnp.float32)

    return pl.pallas_call(
        body,
        out_shape=jax.ShapeDtypeStruct((2, h.shape[0], RW), jnp.float32),
    )(h, wre)


def _tc_sum_bias_proj(msgp, gb, wre):
    def body(mp_ref, b_ref, w_ref, o_ref):
        h = lax.concatenate([mp_ref[0], mp_ref[1]], 1) + b_ref[...]
        o_ref[0] = lax.dot_general(h, w_ref[0], (((1,), (1,)), ((), ())),
                                   preferred_element_type=jnp.float32)
        o_ref[1] = lax.dot_general(h, w_ref[1], (((1,), (1,)), ((), ())),
                                   preferred_element_type=jnp.float32)

    return pl.pallas_call(
        body,
        out_shape=jax.ShapeDtypeStruct((2, NP, RW), jnp.float32),
    )(msgp, gb.reshape(1, F), wre)


def _tc_final(msgp, gb):
    def body(mp_ref, b_ref, o_ref):
        o_ref[...] = lax.concatenate([mp_ref[0], mp_ref[1]], 1) + b_ref[...]

    return pl.pallas_call(
        body,
        out_shape=jax.ShapeDtypeStruct((N, F), jnp.float32),
    )(msgp[:, :N, :], gb.reshape(1, F))


def _pack_params(pp_W, pp_b, mu, inv_sigma):
    scal = jnp.concatenate([
        pp_W[:, 0], pp_W[:, 1], pp_b,
        mu.reshape(-1), inv_sigma.reshape(-1),
    ]).astype(jnp.float32)
    return jnp.broadcast_to(scal[:, None], (18, L))


def _reorder_w(fc_W):
    return fc_W.reshape(K, 2, HF, F).transpose(1, 0, 2, 3).reshape(2, RW, F)


def kernel(x, pp_W1, pp_b1, fc_W1, mu1, inv_sigma1, gbias1,
           pp_W2, pp_b2, fc_W2, mu2, inv_sigma2, gbias2, edge_index):
    src3 = edge_index[0].reshape(16, NCH, CH)
    dst3 = edge_index[1].reshape(16, NCH, CH)
    dst3_deg = edge_index[1].reshape(32, E // (32 * CHD), CHD)
    degp = _sc_degree(dst3_deg)
    rsq = _tc_rsq(degp)
    proj1 = _tc_proj(x, _reorder_w(fc_W1))
    msgp1 = _sc_msg(proj1, rsq, src3, dst3,
                    _pack_params(pp_W1, pp_b1, mu1, inv_sigma1))
    proj2 = _tc_sum_bias_proj(msgp1, gbias1, _reorder_w(fc_W2))
    msgp2 = _sc_msg(proj2, rsq, src3, dst3,
                    _pack_params(pp_W2, pp_b2, mu2, inv_sigma2))
    return _tc_final(msgp2, gbias2)


# speedup vs baseline: 11.8727x; 1.0839x over previous
import dataclasses
import functools

import numpy as np

import jax
import jax.numpy as jnp
from jax import lax
from jax.experimental import pallas as pl
from jax.experimental.pallas import tpu as pltpu
from jax.experimental.pallas import tpu_sc as plsc

N = 10000
NP = 10240
E = 320000
F = 128
HF = F // 2
K = 3
D = 2
L = 16
RW = K * HF
CH = 32
CHD = 16
EPT = E // 16
NCH = EPT // CH
STRIPE = NP // 16


def _cparams():
    cp = pltpu.CompilerParams()
    fields = pltpu.CompilerParams.__dataclass_fields__
    if "needs_layout_passes" in fields:
        cp = dataclasses.replace(cp, needs_layout_passes=False)
    if "use_tc_tiling_on_sc" in fields:
        cp = dataclasses.replace(cp, use_tc_tiling_on_sc=False)
    return cp


def _vc(v, dtype=jnp.float32):
    return jnp.full((L,), v, dtype)


def _sc_degree(dst3):
    nch = dst3.shape[1]
    mesh = plsc.VectorSubcoreMesh(core_axis_name="c", subcore_axis_name="s")

    @functools.partial(
        pl.kernel,
        out_type=jax.ShapeDtypeStruct((2, NP), jnp.float32),
        mesh=mesh,
        compiler_params=_cparams(),
        scratch_types=[
            pltpu.VMEM((nch, CHD), jnp.int32),
            pltpu.VMEM((L,), jnp.float32),
            pltpu.VMEM((L,), jnp.float32),
            pltpu.VMEM_SHARED((NP,), jnp.float32),
        ],
    )
    def kfn(dst_hbm, deg_hbm, dst_v, one_v, zer_v, deg_sh):
        cid = lax.axis_index("c")
        sid = lax.axis_index("s")
        wid = cid * 16 + sid
        pltpu.sync_copy(dst_hbm.at[wid], dst_v)
        one_v[...] = _vc(1.0)
        zer_v[...] = _vc(0.0)

        @pl.loop(0, STRIPE // L)
        def _(t):
            pltpu.sync_copy(zer_v, deg_sh.at[pl.ds(sid * STRIPE + t * L, L)])

        plsc.subcore_barrier()

        @pl.loop(0, nch)
        def _(c):
            pltpu.sync_copy(one_v, deg_sh.at[dst_v.at[c]], add=True)

        plsc.subcore_barrier()
        pltpu.sync_copy(deg_sh.at[pl.ds(sid * STRIPE, STRIPE)],
                        deg_hbm.at[cid, pl.ds(sid * STRIPE, STRIPE)])

    return kfn(dst3)


def _sc_msg(proj2h, rsq, src3, dst3, par):
    mesh = plsc.VectorSubcoreMesh(core_axis_name="c", subcore_axis_name="s")

    @functools.partial(
        pl.kernel,
        out_type=jax.ShapeDtypeStruct((2, NP, HF), jnp.float32),
        mesh=mesh,
        compiler_params=_cparams(),
        scratch_types=[
            pltpu.VMEM((NCH, CH), jnp.int32),
            pltpu.VMEM((NCH, CH), jnp.int32),
            pltpu.VMEM((NP,), jnp.float32),
            pltpu.VMEM((18, L), jnp.float32),
            pltpu.VMEM((2, CH, RW), jnp.bfloat16),
            pltpu.VMEM((2, CH, HF), jnp.float32),
            pltpu.VMEM((2, K, L), jnp.float32),
            pltpu.VMEM_SHARED((NP, HF), jnp.float32),
            pltpu.SemaphoreType.DMA,
            pltpu.SemaphoreType.DMA,
            pltpu.SemaphoreType.DMA,
            pltpu.SemaphoreType.DMA,
        ],
    )
    def kfn(proj_hbm, rsq_hbm, src_hbm, dst_hbm, par_hbm, out_hbm,
            src_v, dst_v, rsq_v, par_v, row_v, msg_v, gw_v, agg_sh,
            sem0, sem1, sct0, sct1):
        cid = lax.axis_index("c")
        sid = lax.axis_index("s")
        pltpu.sync_copy(src_hbm.at[sid], src_v)
        pltpu.sync_copy(dst_hbm.at[sid], dst_v)
        pltpu.sync_copy(rsq_hbm, rsq_v)
        pltpu.sync_copy(par_hbm, par_v)

        for i in range(CH):
            for j in range(HF // L):
                msg_v[0, i, pl.ds(j * L, L)] = _vc(0.0)

        @pl.loop(0, STRIPE // CH)
        def _(t):
            pltpu.sync_copy(msg_v.at[0],
                            agg_sh.at[pl.ds(sid * STRIPE + t * CH, CH)])

        plsc.subcore_barrier()

        pA = [par_v[0, :], par_v[1, :]]
        pB = [par_v[2, :], par_v[3, :]]
        pC = [par_v[4, :], par_v[5, :]]
        pMU = [[par_v[6 + 2 * k + d_, :] for d_ in range(D)] for k in range(K)]
        pIS = [[par_v[12 + 2 * k + d_, :] for d_ in range(D)] for k in range(K)]

        def compute_chunk(c, buf):
            for g in range(CH // L):
                s16 = src_v[c, pl.ds(g * L, L)]
                d16 = dst_v[c, pl.ds(g * L, L)]
                ru = plsc.load_gather(rsq_v, [s16])
                rv = plsc.load_gather(rsq_v, [d16])
                ps = []
                for d_ in range(D):
                    z = pA[d_] * ru + pB[d_] * rv + pC[d_]
                    az = jnp.abs(z)
                    e2 = jnp.exp(az + az)
                    t = _vc(1.0) - _vc(2.0) / (e2 + _vc(1.0))
                    ps.append(jnp.sign(z) * t)
                for k in range(K):
                    u0 = (ps[0] - pMU[k][0]) * pIS[k][0]
                    u1 = (ps[1] - pMU[k][1]) * pIS[k][1]
                    gw_v[g, k, :] = jnp.exp(_vc(-0.5) * (u0 * u0 + u1 * u1))
                for i in range(L):
                    ii = jnp.full((L,), i, jnp.int32)
                    bw = []
                    for k in range(K):
                        gk = plsc.load_gather(gw_v.at[g, k], [ii])
                        bw.append(plsc.pack(gk, gk,
                                            format=plsc.PackFormat.INTERLEAVED))
                    e = g * L + i
                    for G in range(HF // 32):
                        acc = None
                        for k in range(K):
                            v = row_v[buf, e, pl.ds(k * HF + G * 32, 32)]
                            term = v * bw[k]
                            acc = term if acc is None else acc + term
                        lo, hi = plsc.unpack(
                            acc, format=plsc.PackFormat.INTERLEAVED)
                        msg_v[buf, e, pl.ds(G * 32, L)] = lo
                        msg_v[buf, e, pl.ds(G * 32 + L, L)] = hi

        def gather(c, buf, sem):
            pltpu.async_copy(proj_hbm.at[cid].at[src_v.at[c]],
                             row_v.at[buf], sem)

        def wait_gather(c, buf, sem):
            pltpu.make_async_copy(proj_hbm.at[cid].at[src_v.at[c]],
                                  row_v.at[buf], sem).wait()

        def scatter(c, buf, sem):
            pltpu.async_copy(msg_v.at[buf], agg_sh.at[dst_v.at[c]], sem,
                             add=True)

        def wait_scatter(c, buf, sem):
            pltpu.make_async_copy(msg_v.at[buf], agg_sh.at[dst_v.at[c]],
                                  sem).wait()

        gather(0, 0, sem0)
        gather(1, 1, sem1)

        @pl.loop(0, NCH // 2)
        def _(p):
            c0 = p * 2
            wait_gather(c0, 0, sem0)

            @pl.when(p >= 1)
            def _():
                wait_scatter(c0 - 2, 0, sct0)

            compute_chunk(c0, 0)
            scatter(c0, 0, sct0)
            gather(c0 + 2, 0, sem0)
            wait_gather(c0 + 1, 1, sem1)

            @pl.when(p >= 1)
            def _():
                wait_scatter(c0 - 1, 1, sct1)

            compute_chunk(c0 + 1, 1)
            scatter(c0 + 1, 1, sct1)

            @pl.when(p < NCH // 2 - 1)
            def _():
                gather(c0 + 3, 1, sem1)

        wait_gather(NCH - 1, 0, sem0)
        wait_scatter(NCH - 3, 0, sct0)
        compute_chunk(NCH - 1, 0)
        scatter(NCH - 1, 0, sct0)
        wait_scatter(NCH - 2, 1, sct1)
        wait_scatter(NCH - 1, 0, sct0)

        plsc.subcore_barrier()

        @pl.loop(0, STRIPE // 128)
        def _(t):
            pltpu.sync_copy(agg_sh.at[pl.ds(sid * STRIPE + t * 128, 128)],
                            out_hbm.at[cid, pl.ds(sid * STRIPE + t * 128, 128)])

    return kfn(proj2h, rsq, src3, dst3, par)


def _tc_rsq(degp):
    def body(dp_ref, o_ref):
        o_ref[...] = lax.rsqrt(dp_ref[0] + dp_ref[1])

    out = pl.pallas_call(
        body,
        out_shape=jax.ShapeDtypeStruct((NP // 128, 128), jnp.float32),
    )(degp.reshape(2, NP // 128, 128))
    return out.reshape(NP)


def _tc_proj(h, wre):
    def body(h_ref, w_ref, o_ref):
        hv = h_ref[...]
        for c in range(2):
            o_ref[c] = lax.dot_general(
                hv, w_ref[c], (((1,), (1,)), ((), ())),
                preferred_element_type=jnp.float32).astype(jnp.bfloat16)

    return pl.pallas_call(
        body,
        out_shape=jax.ShapeDtypeStruct((2, h.shape[0], RW), jnp.bfloat16),
    )(h, wre)


def _tc_sum_bias_proj(msgp, gb, wre):
    def body(mp_ref, b_ref, w_ref, o_ref):
        h = lax.concatenate([mp_ref[0], mp_ref[1]], 1) + b_ref[...]
        for c in range(2):
            o_ref[c] = lax.dot_general(
                h, w_ref[c], (((1,), (1,)), ((), ())),
                preferred_element_type=jnp.float32).astype(jnp.bfloat16)

    return pl.pallas_call(
        body,
        out_shape=jax.ShapeDtypeStruct((2, NP, RW), jnp.bfloat16),
    )(msgp, gb.reshape(1, F), wre)


def _tc_final(msgp, gb):
    def body(mp_ref, b_ref, o_ref):
        o_ref[...] = lax.concatenate([mp_ref[0], mp_ref[1]], 1) + b_ref[...]

    return pl.pallas_call(
        body,
        out_shape=jax.ShapeDtypeStruct((N, F), jnp.float32),
    )(msgp[:, :N, :], gb.reshape(1, F))


def _pack_params(pp_W, pp_b, mu, inv_sigma):
    scal = jnp.concatenate([
        pp_W[:, 0], pp_W[:, 1], pp_b,
        mu.reshape(-1), inv_sigma.reshape(-1),
    ]).astype(jnp.float32)
    return jnp.broadcast_to(scal[:, None], (18, L))


_PERM64 = np.concatenate([
    G * 32 + np.array([(l >> 1) + 16 * (l & 1) for l in range(32)])
    for G in range(2)
])
_RWPERM = np.concatenate([k * HF + _PERM64 for k in range(K)])


def _reorder_w(fc_W):
    wre = fc_W.reshape(K, 2, HF, F).transpose(1, 0, 2, 3).reshape(2, RW, F)
    return wre[:, _RWPERM, :]


def kernel(x, pp_W1, pp_b1, fc_W1, mu1, inv_sigma1, gbias1,
           pp_W2, pp_b2, fc_W2, mu2, inv_sigma2, gbias2, edge_index):
    src3 = edge_index[0].reshape(16, NCH, CH)
    dst3 = edge_index[1].reshape(16, NCH, CH)
    dst3_deg = edge_index[1].reshape(32, E // (32 * CHD), CHD)
    degp = _sc_degree(dst3_deg)
    rsq = _tc_rsq(degp)
    proj1 = _tc_proj(x, _reorder_w(fc_W1))
    msgp1 = _sc_msg(proj1, rsq, src3, dst3,
                    _pack_params(pp_W1, pp_b1, mu1, inv_sigma1))
    proj2 = _tc_sum_bias_proj(msgp1, gbias1, _reorder_w(fc_W2))
    msgp2 = _sc_msg(proj2, rsq, src3, dst3,
                    _pack_params(pp_W2, pp_b2, mu2, inv_sigma2))
    return _tc_final(msgp2, gbias2)
